# baseline (speedup 1.0000x reference)
import functools

import jax
import jax.numpy as jnp
from jax import lax
from jax.experimental import pallas as pl
from jax.experimental.pallas import tpu as pltpu
from jax.experimental.pallas import tpu_sc as plsc

B = 4096
F = 26
D = 32
NFLAT = B * F
NC, NS = 2, 16
NW = NC * NS
PER_W = NFLAT // NW

BN_EPS = 1e-5

_mesh = plsc.VectorSubcoreMesh(core_axis_name="c", subcore_axis_name="s")


@functools.partial(
    pl.kernel,
    mesh=_mesh,
    out_type=jax.ShapeDtypeStruct((NFLAT,), jnp.float32),
    scratch_types=[
        pltpu.VMEM((PER_W,), jnp.int32),
        pltpu.VMEM((PER_W,), jnp.float32),
        pltpu.SemaphoreType.DMA,
    ],
    compiler_params=pltpu.CompilerParams(use_tc_tiling_on_sc=False),
)
def _sc_o1_gather(idx_hbm, o1_tab, o1_out, idx_v, o1_v, sem):
    wid = lax.axis_index("s") * NC + lax.axis_index("c")
    base = wid * PER_W
    pltpu.sync_copy(idx_hbm.at[pl.ds(base, PER_W)], idx_v)
    pltpu.async_copy(o1_tab.at[idx_v], o1_v, sem).wait()
    pltpu.sync_copy(o1_v, o1_out.at[pl.ds(base, PER_W)])


def _tc_stage1(emb_ref, o1v_ref, W1_ref, b1_ref, g1_ref, bt1_ref,
               h1_ref, o12_ref):
    emb = emb_ref[...]
    s = emb[:, 0:D]
    sq = emb * emb
    for f in range(1, F):
        s = s + emb[:, f * D:(f + 1) * D]
    s = s.astype(jnp.float32)
    sq_of_sum = jnp.sum(s * s, axis=1, keepdims=True)
    sum_of_sq = jnp.sum(sq.astype(jnp.float32), axis=1, keepdims=True)
    o2 = 0.5 * (sq_of_sum - sum_of_sq)
    o1 = jnp.sum(o1v_ref[...], axis=1, keepdims=True)
    o12_ref[...] = o1 + o2
    h = jnp.dot(emb, W1_ref[...], preferred_element_type=jnp.float32) + b1_ref[...]
    mu = jnp.mean(h, axis=0, keepdims=True)
    var = jnp.mean((h - mu) ** 2, axis=0, keepdims=True)
    h = (h - mu) / jnp.sqrt(var + BN_EPS) * g1_ref[...] + bt1_ref[...]
    h1_ref[...] = jnp.maximum(h, 0.0).astype(jnp.bfloat16)


def _tc_stage2(h1_ref, o12_ref, W2_ref, b2_ref, g2_ref, bt2_ref,
               W3_ref, b3_ref, W4_ref, b4_ref, out_ref):
    h1 = h1_ref[...]
    h = jnp.dot(h1, W2_ref[...], preferred_element_type=jnp.float32) + b2_ref[...]
    mu = jnp.mean(h, axis=0, keepdims=True)
    var = jnp.mean((h - mu) ** 2, axis=0, keepdims=True)
    h = (h - mu) / jnp.sqrt(var + BN_EPS) * g2_ref[...] + bt2_ref[...]
    h = jnp.maximum(h, 0.0).astype(jnp.bfloat16)
    h = jnp.dot(h, W3_ref[...], preferred_element_type=jnp.float32) + b3_ref[...]
    dnn = jnp.dot(h.astype(jnp.bfloat16), W4_ref[...],
                  preferred_element_type=jnp.float32) + b4_ref[...]
    out_ref[...] = o12_ref[...] + dnn


def kernel(x, cat_embed, o1_table, W1, b1, g1, bt1, W2, b2, g2, bt2,
           W3, b3, W4, b4):
    bf = jnp.bfloat16
    idx = x.reshape(-1).astype(jnp.int32)
    o1_flat = _sc_o1_gather(idx, o1_table[:, 0])
    emb2d = jnp.take(cat_embed, idx, axis=0).reshape(B, F * D).astype(bf)
    o1v = o1_flat.reshape(B, F)
    h1, o12 = pl.pallas_call(
        _tc_stage1,
        out_shape=(jax.ShapeDtypeStruct((B, 1024), bf),
                   jax.ShapeDtypeStruct((B, 1), jnp.float32)),
    )(emb2d, o1v, W1.astype(bf), b1.reshape(1, -1), g1.reshape(1, -1),
      bt1.reshape(1, -1))
    out = pl.pallas_call(
        _tc_stage2,
        out_shape=jax.ShapeDtypeStruct((B, 1), jnp.float32),
    )(h1, o12, W2.astype(bf), b2.reshape(1, -1), g2.reshape(1, -1),
      bt2.reshape(1, -1), W3.astype(bf), b3.reshape(1, -1),
      W4.astype(bf), b4.reshape(1, -1))
    return out


# speedup vs baseline: 6.6028x; 6.6028x over previous
import functools

import jax
import jax.numpy as jnp
from jax import lax
from jax.experimental import pallas as pl
from jax.experimental.pallas import tpu as pltpu
from jax.experimental.pallas import tpu_sc as plsc

B = 4096
F = 26
D = 32
NFLAT = B * F
NC, NS = 2, 16
NW = NC * NS
PER_W = NFLAT // NW

BN_EPS = 1e-5

_mesh = plsc.VectorSubcoreMesh(core_axis_name="c", subcore_axis_name="s")


@functools.partial(
    pl.kernel,
    mesh=_mesh,
    out_type=jax.ShapeDtypeStruct((NFLAT,), jnp.float32),
    scratch_types=[
        pltpu.VMEM((PER_W,), jnp.int32),
        pltpu.VMEM((PER_W,), jnp.float32),
        pltpu.SemaphoreType.DMA,
    ],
    compiler_params=pltpu.CompilerParams(use_tc_tiling_on_sc=False),
)
def _sc_o1_gather(idx_hbm, o1_tab, o1_out, idx_v, o1_v, sem):
    wid = lax.axis_index("s") * NC + lax.axis_index("c")
    base = wid * PER_W
    pltpu.sync_copy(idx_hbm.at[pl.ds(base, PER_W)], idx_v)
    pltpu.async_copy(o1_tab.at[idx_v], o1_v, sem).wait()
    pltpu.sync_copy(o1_v, o1_out.at[pl.ds(base, PER_W)])


def _tc_fm_cast(emb_ref, o1v_ref, embbf_ref, o12_ref):
    s = emb_ref[:, 0:D]
    sum_of_sq = jnp.sum(s * s, axis=1, keepdims=True)
    for f in range(1, F):
        c = emb_ref[:, f * D:(f + 1) * D]
        s = s + c
        sum_of_sq = sum_of_sq + jnp.sum(c * c, axis=1, keepdims=True)
    sq_of_sum = jnp.sum(s * s, axis=1, keepdims=True)
    o2 = 0.5 * (sq_of_sum - sum_of_sq)
    o1 = jnp.sum(o1v_ref[...], axis=1, keepdims=True)
    o12_ref[...] = o1 + o2
    embbf_ref[...] = emb_ref[...].astype(jnp.bfloat16)


def _tc_mm_bn_relu(x_ref, W_ref, b_ref, g_ref, bt_ref, out_ref):
    h = jnp.dot(x_ref[...], W_ref[...],
                preferred_element_type=jnp.float32) + b_ref[...]
    mu = jnp.mean(h, axis=0, keepdims=True)
    var = jnp.mean((h - mu) ** 2, axis=0, keepdims=True)
    h = (h - mu) / jnp.sqrt(var + BN_EPS) * g_ref[...] + bt_ref[...]
    out_ref[...] = jnp.maximum(h, 0.0).astype(jnp.bfloat16)


def _tc_tail(h2_ref, o12_ref, W3_ref, b3_ref, W4_ref, b4_ref, out_ref):
    h3 = jnp.dot(h2_ref[...], W3_ref[...],
                 preferred_element_type=jnp.float32) + b3_ref[...]
    dnn = jnp.dot(h3.astype(jnp.bfloat16), W4_ref[...],
                  preferred_element_type=jnp.float32) + b4_ref[...]
    out_ref[...] = o12_ref[...] + dnn


def _mm_bn_relu_call(x, W, b, g, bt, n_out, blk):
    grid = (n_out // blk,)
    d_in = x.shape[1]
    return pl.pallas_call(
        _tc_mm_bn_relu,
        grid=grid,
        in_specs=[
            pl.BlockSpec((B, d_in), lambda j: (0, 0)),
            pl.BlockSpec((d_in, blk), lambda j: (0, j)),
            pl.BlockSpec((1, blk), lambda j: (0, j)),
            pl.BlockSpec((1, blk), lambda j: (0, j)),
            pl.BlockSpec((1, blk), lambda j: (0, j)),
        ],
        out_specs=pl.BlockSpec((B, blk), lambda j: (0, j)),
        out_shape=jax.ShapeDtypeStruct((B, n_out), jnp.bfloat16),
    )(x, W, b.reshape(1, -1), g.reshape(1, -1), bt.reshape(1, -1))


def kernel(x, cat_embed, o1_table, W1, b1, g1, bt1, W2, b2, g2, bt2,
           W3, b3, W4, b4):
    bf = jnp.bfloat16
    idx = x.reshape(-1).astype(jnp.int32)
    o1_flat = _sc_o1_gather(idx, o1_table[:, 0])
    emb2d = jnp.take(cat_embed, idx, axis=0).reshape(B, F * D)
    o1v = o1_flat.reshape(B, F)

    BB = 512
    embbf, o12 = pl.pallas_call(
        _tc_fm_cast,
        grid=(B // BB,),
        in_specs=[
            pl.BlockSpec((BB, F * D), lambda i: (i, 0)),
            pl.BlockSpec((BB, F), lambda i: (i, 0)),
        ],
        out_specs=(
            pl.BlockSpec((BB, F * D), lambda i: (i, 0)),
            pl.BlockSpec((BB, 1), lambda i: (i, 0)),
        ),
        out_shape=(jax.ShapeDtypeStruct((B, F * D), bf),
                   jax.ShapeDtypeStruct((B, 1), jnp.float32)),
    )(emb2d, o1v)

    h1 = _mm_bn_relu_call(embbf, W1.astype(bf), b1, g1, bt1, 1024, 256)
    h2 = _mm_bn_relu_call(h1, W2.astype(bf), b2, g2, bt2, 512, 256)
    out = pl.pallas_call(
        _tc_tail,
        out_shape=jax.ShapeDtypeStruct((B, 1), jnp.float32),
    )(h2, o12, W3.astype(bf), b3.reshape(1, -1),
      W4.astype(bf), b4.reshape(1, -1))
    return out


# speedup vs baseline: 6.9067x; 1.0460x over previous
import functools

import jax
import jax.numpy as jnp
from jax import lax
from jax.experimental import pallas as pl
from jax.experimental.pallas import tpu as pltpu
from jax.experimental.pallas import tpu_sc as plsc

B = 4096
F = 26
D = 32
NFLAT = B * F
NC, NS = 2, 16
NW = NC * NS
PER_W = NFLAT // NW

BN_EPS = 1e-5

_mesh = plsc.VectorSubcoreMesh(core_axis_name="c", subcore_axis_name="s")


@functools.partial(
    pl.kernel,
    mesh=_mesh,
    out_type=jax.ShapeDtypeStruct((NFLAT,), jnp.float32),
    scratch_types=[
        pltpu.VMEM((PER_W,), jnp.int32),
        pltpu.VMEM((PER_W,), jnp.float32),
        pltpu.SemaphoreType.DMA,
    ],
    compiler_params=pltpu.CompilerParams(use_tc_tiling_on_sc=False),
)
def _sc_o1_gather(idx_hbm, o1_tab, o1_out, idx_v, o1_v, sem):
    wid = lax.axis_index("s") * NC + lax.axis_index("c")
    base = wid * PER_W
    pltpu.sync_copy(idx_hbm.at[pl.ds(base, PER_W)], idx_v)
    pltpu.async_copy(o1_tab.at[idx_v], o1_v, sem).wait()
    pltpu.sync_copy(o1_v, o1_out.at[pl.ds(base, PER_W)])


BB = 512
BLK1 = 256
BLK2 = 256


def _tc_body(emb_ref, o1v_ref, W1_ref, b1_ref, g1_ref, bt1_ref,
             W2_ref, b2_ref, g2_ref, bt2_ref, W3_ref, b3_ref,
             W4_ref, b4_ref, out_ref, embbf_s, h1_s, h2_s):
    bf = jnp.bfloat16
    for i in range(B // BB):
        r = pl.ds(i * BB, BB)
        s = emb_ref[r, 0:D]
        ssq = jnp.sum(s * s, axis=1, keepdims=True)
        for f in range(1, F):
            c = emb_ref[r, f * D:(f + 1) * D]
            s = s + c
            ssq = ssq + jnp.sum(c * c, axis=1, keepdims=True)
        o2 = 0.5 * (jnp.sum(s * s, axis=1, keepdims=True) - ssq)
        o1 = jnp.sum(o1v_ref[r, :], axis=1, keepdims=True)
        out_ref[r, :] = o1 + o2
        embbf_s[r, :] = emb_ref[r, :].astype(bf)

    for j in range(1024 // BLK1):
        cbl = pl.ds(j * BLK1, BLK1)
        h = jnp.dot(embbf_s[...], W1_ref[:, cbl],
                    preferred_element_type=jnp.float32) + b1_ref[:, cbl]
        mu = jnp.mean(h, axis=0, keepdims=True)
        var = jnp.mean((h - mu) ** 2, axis=0, keepdims=True)
        h = (h - mu) / jnp.sqrt(var + BN_EPS) * g1_ref[:, cbl] + bt1_ref[:, cbl]
        h1_s[:, cbl] = jnp.maximum(h, 0.0).astype(bf)

    for j in range(512 // BLK2):
        cbl = pl.ds(j * BLK2, BLK2)
        h = jnp.dot(h1_s[...], W2_ref[:, cbl],
                    preferred_element_type=jnp.float32) + b2_ref[:, cbl]
        mu = jnp.mean(h, axis=0, keepdims=True)
        var = jnp.mean((h - mu) ** 2, axis=0, keepdims=True)
        h = (h - mu) / jnp.sqrt(var + BN_EPS) * g2_ref[:, cbl] + bt2_ref[:, cbl]
        h2_s[:, cbl] = jnp.maximum(h, 0.0).astype(bf)

    h3 = jnp.dot(h2_s[...], W3_ref[...],
                 preferred_element_type=jnp.float32) + b3_ref[...]
    dnn = jnp.dot(h3.astype(bf), W4_ref[...],
                  preferred_element_type=jnp.float32) + b4_ref[...]
    out_ref[...] = out_ref[...] + dnn


def kernel(x, cat_embed, o1_table, W1, b1, g1, bt1, W2, b2, g2, bt2,
           W3, b3, W4, b4):
    bf = jnp.bfloat16
    idx = x.reshape(-1).astype(jnp.int32)
    o1_flat = _sc_o1_gather(idx, o1_table[:, 0])
    emb2d = jnp.take(cat_embed, idx, axis=0).reshape(B, F * D)
    o1v = o1_flat.reshape(B, F)
    out = pl.pallas_call(
        _tc_body,
        out_shape=jax.ShapeDtypeStruct((B, 1), jnp.float32),
        scratch_shapes=[
            pltpu.VMEM((B, F * D), bf),
            pltpu.VMEM((B, 1024), bf),
            pltpu.VMEM((B, 512), bf),
        ],
    )(emb2d, o1v, W1.astype(bf), b1.reshape(1, -1), g1.reshape(1, -1),
      bt1.reshape(1, -1), W2.astype(bf), b2.reshape(1, -1),
      g2.reshape(1, -1), bt2.reshape(1, -1), W3.astype(bf),
      b3.reshape(1, -1), W4.astype(bf), b4.reshape(1, -1))
    return out


# speedup vs baseline: 7.1508x; 1.0353x over previous
import functools

import jax
import jax.numpy as jnp
from jax import lax
from jax.experimental import pallas as pl
from jax.experimental.pallas import tpu as pltpu
from jax.experimental.pallas import tpu_sc as plsc

B = 4096
F = 26
D = 32
NFLAT = B * F
NC, NS = 2, 16
NW = NC * NS
PER_W = NFLAT // NW

BN_EPS = 1e-5

_mesh = plsc.VectorSubcoreMesh(core_axis_name="c", subcore_axis_name="s")


@functools.partial(
    pl.kernel,
    mesh=_mesh,
    out_type=jax.ShapeDtypeStruct((NFLAT,), jnp.float32),
    scratch_types=[
        pltpu.VMEM((PER_W,), jnp.int32),
        pltpu.VMEM((PER_W,), jnp.float32),
        pltpu.SemaphoreType.DMA,
    ],
    compiler_params=pltpu.CompilerParams(use_tc_tiling_on_sc=False),
)
def _sc_o1_gather(idx_hbm, o1_tab, o1_out, idx_v, o1_v, sem):
    wid = lax.axis_index("s") * NC + lax.axis_index("c")
    base = wid * PER_W
    pltpu.sync_copy(idx_hbm.at[pl.ds(base, PER_W)], idx_v)
    pltpu.async_copy(o1_tab.at[idx_v], o1_v, sem).wait()
    pltpu.sync_copy(o1_v, o1_out.at[pl.ds(base, PER_W)])


BB = 512
BLK1 = 256
BLK2 = 256


def _bn_relu(h, g, bt, ones_b):
    inv_b = 1.0 / B
    mu = jnp.dot(ones_b, h, preferred_element_type=jnp.float32) * inv_b
    m2 = jnp.dot(ones_b, h * h, preferred_element_type=jnp.float32) * inv_b
    var = m2 - mu * mu
    h = (h - mu) / jnp.sqrt(var + BN_EPS) * g + bt
    return jnp.maximum(h, 0.0)


def _tc_body(emb_ref, o1v_ref, W1_ref, b1_ref, g1_ref, bt1_ref,
             W2_ref, b2_ref, g2_ref, bt2_ref, W3_ref, b3_ref,
             W4_ref, b4_ref, out_ref, embbf_s, h1_s, h2_s):
    bf = jnp.bfloat16
    ones_b = jnp.ones((1, B), jnp.float32)
    rows = jax.lax.broadcasted_iota(jnp.int32, (F * D, D), 0)
    cols = jax.lax.broadcasted_iota(jnp.int32, (F * D, D), 1)
    S = jnp.where(rows % D == cols, 1.0, 0.0).astype(jnp.float32)
    ones_fd = jnp.ones((F * D, 1), jnp.float32)
    ones_d = jnp.ones((D, 1), jnp.float32)
    ones_f = jnp.ones((F, 1), jnp.float32)

    for i in range(B // BB):
        r = pl.ds(i * BB, BB)
        e = emb_ref[r, :]
        s = jnp.dot(e, S, preferred_element_type=jnp.float32)
        ssq = jnp.dot(e * e, ones_fd, preferred_element_type=jnp.float32)
        sqs = jnp.dot(s * s, ones_d, preferred_element_type=jnp.float32)
        o1 = jnp.dot(o1v_ref[r, :], ones_f, preferred_element_type=jnp.float32)
        out_ref[r, :] = o1 + 0.5 * (sqs - ssq)
        embbf_s[r, :] = e.astype(bf)

    for j in range(1024 // BLK1):
        cbl = pl.ds(j * BLK1, BLK1)
        h = jnp.dot(embbf_s[...], W1_ref[:, cbl],
                    preferred_element_type=jnp.float32) + b1_ref[:, cbl]
        h1_s[:, cbl] = _bn_relu(h, g1_ref[:, cbl], bt1_ref[:, cbl],
                                ones_b).astype(bf)

    for j in range(512 // BLK2):
        cbl = pl.ds(j * BLK2, BLK2)
        h = jnp.dot(h1_s[...], W2_ref[:, cbl],
                    preferred_element_type=jnp.float32) + b2_ref[:, cbl]
        h2_s[:, cbl] = _bn_relu(h, g2_ref[:, cbl], bt2_ref[:, cbl],
                                ones_b).astype(bf)

    h3 = jnp.dot(h2_s[...], W3_ref[...],
                 preferred_element_type=jnp.float32) + b3_ref[...]
    dnn = jnp.dot(h3.astype(bf), W4_ref[...],
                  preferred_element_type=jnp.float32) + b4_ref[...]
    out_ref[...] = out_ref[...] + dnn


def kernel(x, cat_embed, o1_table, W1, b1, g1, bt1, W2, b2, g2, bt2,
           W3, b3, W4, b4):
    bf = jnp.bfloat16
    idx = x.reshape(-1).astype(jnp.int32)
    o1_flat = _sc_o1_gather(idx, o1_table[:, 0])
    emb2d = jnp.take(cat_embed, idx, axis=0).reshape(B, F * D)
    o1v = o1_flat.reshape(B, F)
    out = pl.pallas_call(
        _tc_body,
        out_shape=jax.ShapeDtypeStruct((B, 1), jnp.float32),
        scratch_shapes=[
            pltpu.VMEM((B, F * D), bf),
            pltpu.VMEM((B, 1024), bf),
            pltpu.VMEM((B, 512), bf),
        ],
    )(emb2d, o1v, W1.astype(bf), b1.reshape(1, -1), g1.reshape(1, -1),
      bt1.reshape(1, -1), W2.astype(bf), b2.reshape(1, -1),
      g2.reshape(1, -1), bt2.reshape(1, -1), W3.astype(bf),
      b3.reshape(1, -1), W4.astype(bf), b4.reshape(1, -1))
    return out


# speedup vs baseline: 7.1667x; 1.0022x over previous
import functools

import jax
import jax.numpy as jnp
from jax import lax
from jax.experimental import pallas as pl
from jax.experimental.pallas import tpu as pltpu
from jax.experimental.pallas import tpu_sc as plsc

B = 4096
F = 26
D = 32
NFLAT = B * F
NC, NS = 2, 16
NW = NC * NS
PER_W = NFLAT // NW

BN_EPS = 1e-5

_mesh = plsc.VectorSubcoreMesh(core_axis_name="c", subcore_axis_name="s")


@functools.partial(
    pl.kernel,
    mesh=_mesh,
    out_type=jax.ShapeDtypeStruct((NFLAT,), jnp.float32),
    scratch_types=[
        pltpu.VMEM((PER_W,), jnp.int32),
        pltpu.VMEM((PER_W,), jnp.float32),
        pltpu.SemaphoreType.DMA,
    ],
    compiler_params=pltpu.CompilerParams(use_tc_tiling_on_sc=False),
)
def _sc_o1_gather(idx_hbm, o1_tab, o1_out, idx_v, o1_v, sem):
    wid = lax.axis_index("s") * NC + lax.axis_index("c")
    base = wid * PER_W
    pltpu.sync_copy(idx_hbm.at[pl.ds(base, PER_W)], idx_v)
    pltpu.async_copy(o1_tab.at[idx_v], o1_v, sem).wait()
    pltpu.sync_copy(o1_v, o1_out.at[pl.ds(base, PER_W)])


BB = 512
BLK1 = 256
BLK2 = 256


def _bn_relu(h, g, bt, ones_b):
    inv_b = 1.0 / B
    mu = jnp.dot(ones_b, h, preferred_element_type=jnp.float32) * inv_b
    m2 = jnp.dot(ones_b, h * h, preferred_element_type=jnp.float32) * inv_b
    var = m2 - mu * mu
    h = (h - mu) / jnp.sqrt(var + BN_EPS) * g + bt
    return jnp.maximum(h, 0.0)


def _tc_body(emb_ref, o1v_ref, W1_ref, b1_ref, g1_ref, bt1_ref,
             W2_ref, b2_ref, g2_ref, bt2_ref, W3_ref, b3_ref,
             W4_ref, b4_ref, out_ref, embbf_s, h1_s, h2_s):
    bf = jnp.bfloat16
    ones_b = jnp.ones((1, B), jnp.float32)
    rows = jax.lax.broadcasted_iota(jnp.int32, (F * D, D), 0)
    cols = jax.lax.broadcasted_iota(jnp.int32, (F * D, D), 1)
    S = jnp.where(rows % D == cols, 1.0, 0.0).astype(jnp.float32)
    ones_fd = jnp.ones((F * D, 1), jnp.float32)
    ones_d = jnp.ones((D, 1), jnp.float32)
    ones_f = jnp.ones((F, 1), jnp.float32)

    for i in range(B // BB):
        r = pl.ds(i * BB, BB)
        e = emb_ref[r, :]
        s = jnp.dot(e, S, preferred_element_type=jnp.float32)
        ssq = jnp.dot(e * e, ones_fd, preferred_element_type=jnp.float32)
        sqs = jnp.dot(s * s, ones_d, preferred_element_type=jnp.float32)
        o1 = jnp.dot(o1v_ref[r, :], ones_f, preferred_element_type=jnp.float32)
        out_ref[r, :] = o1 + 0.5 * (sqs - ssq)
        embbf_s[r, :] = e.astype(bf)

    for j in range(1024 // BLK1):
        cbl = pl.ds(j * BLK1, BLK1)
        h = jnp.dot(embbf_s[...], W1_ref[:, cbl],
                    preferred_element_type=jnp.float32) + b1_ref[:, cbl]
        h1_s[:, cbl] = _bn_relu(h, g1_ref[:, cbl], bt1_ref[:, cbl],
                                ones_b).astype(bf)

    for j in range(512 // BLK2):
        cbl = pl.ds(j * BLK2, BLK2)
        h = jnp.dot(h1_s[...], W2_ref[:, cbl],
                    preferred_element_type=jnp.float32) + b2_ref[:, cbl]
        h2_s[:, cbl] = _bn_relu(h, g2_ref[:, cbl], bt2_ref[:, cbl],
                                ones_b).astype(bf)

    h3 = jnp.dot(h2_s[...], W3_ref[...],
                 preferred_element_type=jnp.float32) + b3_ref[...]
    dnn = jnp.dot(h3.astype(bf), W4_ref[...],
                  preferred_element_type=jnp.float32) + b4_ref[...]
    out_ref[...] = out_ref[...] + dnn


def kernel(x, cat_embed, o1_table, W1, b1, g1, bt1, W2, b2, g2, bt2,
           W3, b3, W4, b4):
    bf = jnp.bfloat16
    idx = x.reshape(-1).astype(jnp.int32)
    emb2d = jnp.take(cat_embed, idx, axis=0).reshape(B, F * D)
    o1_flat = _sc_o1_gather(idx, o1_table[:, 0])
    o1v = o1_flat.reshape(B, F)
    out = pl.pallas_call(
        _tc_body,
        out_shape=jax.ShapeDtypeStruct((B, 1), jnp.float32),
        scratch_shapes=[
            pltpu.VMEM((B, F * D), bf),
            pltpu.VMEM((B, 1024), bf),
            pltpu.VMEM((B, 512), bf),
        ],
    )(emb2d, o1v, W1.astype(bf), b1.reshape(1, -1), g1.reshape(1, -1),
      bt1.reshape(1, -1), W2.astype(bf), b2.reshape(1, -1),
      g2.reshape(1, -1), bt2.reshape(1, -1), W3.astype(bf),
      b3.reshape(1, -1), W4.astype(bf), b4.reshape(1, -1))
    return out


# speedup vs baseline: 10.6997x; 1.4930x over previous
import functools

import jax
import jax.numpy as jnp
from jax import lax
from jax.experimental import pallas as pl
from jax.experimental.pallas import tpu as pltpu
from jax.experimental.pallas import tpu_sc as plsc

B = 4096
F = 26
D = 32
NFLAT = B * F
NC, NS = 2, 16
NW = NC * NS
PER_W = NFLAT // NW

BN_EPS = 1e-5

_mesh = plsc.VectorSubcoreMesh(core_axis_name="c", subcore_axis_name="s")


@functools.partial(
    pl.kernel,
    mesh=_mesh,
    out_type=jax.ShapeDtypeStruct((NFLAT,), jnp.float32),
    scratch_types=[
        pltpu.VMEM((PER_W,), jnp.int32),
        pltpu.VMEM((PER_W,), jnp.float32),
        pltpu.SemaphoreType.DMA,
    ],
    compiler_params=pltpu.CompilerParams(use_tc_tiling_on_sc=False),
)
def _sc_o1_gather(idx_hbm, o1_tab, o1_out, idx_v, o1_v, sem):
    wid = lax.axis_index("s") * NC + lax.axis_index("c")
    base = wid * PER_W
    pltpu.sync_copy(idx_hbm.at[pl.ds(base, PER_W)], idx_v)
    pltpu.async_copy(o1_tab.at[idx_v], o1_v, sem).wait()
    pltpu.sync_copy(o1_v, o1_out.at[pl.ds(base, PER_W)])


BB = 512
BLK1 = 256
BLK2 = 256


def _bn_relu(h, g, bt, ones_b):
    inv_b = 1.0 / B
    mu = jnp.dot(ones_b, h, preferred_element_type=jnp.float32) * inv_b
    m2 = jnp.dot(ones_b, h * h, preferred_element_type=jnp.float32) * inv_b
    var = m2 - mu * mu
    h = (h - mu) / jnp.sqrt(var + BN_EPS) * g + bt
    return jnp.maximum(h, 0.0)


def _tc_body(emb_ref, o1v_ref, W1_ref, b1_ref, g1_ref, bt1_ref,
             W2_ref, b2_ref, g2_ref, bt2_ref, W3_ref, b3_ref,
             W4_ref, b4_ref, out_ref, embbf_s, h1_s, h2_s):
    bf = jnp.bfloat16
    ones_b = jnp.ones((1, B), jnp.float32)
    rows = jax.lax.broadcasted_iota(jnp.int32, (F * D, D), 0)
    cols = jax.lax.broadcasted_iota(jnp.int32, (F * D, D), 1)
    S = jnp.where(rows % D == cols, 1.0, 0.0).astype(jnp.float32)
    ones_fd = jnp.ones((F * D, 1), jnp.float32)
    ones_d = jnp.ones((D, 1), jnp.float32)
    ones_f = jnp.ones((F, 1), jnp.float32)

    for i in range(B // BB):
        r = pl.ds(i * BB, BB)
        e = emb_ref[r, :]
        s = jnp.dot(e, S, preferred_element_type=jnp.float32)
        ssq = jnp.dot(e * e, ones_fd, preferred_element_type=jnp.float32)
        sqs = jnp.dot(s * s, ones_d, preferred_element_type=jnp.float32)
        o1 = jnp.dot(o1v_ref[r, :], ones_f, preferred_element_type=jnp.float32)
        out_ref[r, :] = o1 + 0.5 * (sqs - ssq)
        embbf_s[r, :] = e.astype(bf)

    for j in range(1024 // BLK1):
        cbl = pl.ds(j * BLK1, BLK1)
        h = jnp.dot(embbf_s[...], W1_ref[:, cbl],
                    preferred_element_type=jnp.float32) + b1_ref[:, cbl]
        h1_s[:, cbl] = _bn_relu(h, g1_ref[:, cbl], bt1_ref[:, cbl],
                                ones_b).astype(bf)

    for j in range(512 // BLK2):
        cbl = pl.ds(j * BLK2, BLK2)
        h = jnp.dot(h1_s[...], W2_ref[:, cbl],
                    preferred_element_type=jnp.float32) + b2_ref[:, cbl]
        h2_s[:, cbl] = _bn_relu(h, g2_ref[:, cbl], bt2_ref[:, cbl],
                                ones_b).astype(bf)

    h3 = jnp.dot(h2_s[...], W3_ref[...],
                 preferred_element_type=jnp.float32) + b3_ref[...]
    dnn = jnp.dot(h3.astype(bf), W4_ref[...],
                  preferred_element_type=jnp.float32) + b4_ref[...]
    out_ref[...] = out_ref[...] + dnn


def kernel(x, cat_embed, o1_table, W1, b1, g1, bt1, W2, b2, g2, bt2,
           W3, b3, W4, b4):
    bf = jnp.bfloat16
    idx = x.reshape(-1).astype(jnp.int32)
    emb2d = cat_embed.at[idx].get(mode="promise_in_bounds").reshape(B, F * D)
    o1v = o1_table.at[x].get(mode="promise_in_bounds").reshape(B, F)
    out = pl.pallas_call(
        _tc_body,
        out_shape=jax.ShapeDtypeStruct((B, 1), jnp.float32),
        scratch_shapes=[
            pltpu.VMEM((B, F * D), bf),
            pltpu.VMEM((B, 1024), bf),
            pltpu.VMEM((B, 512), bf),
        ],
    )(emb2d, o1v, W1.astype(bf), b1.reshape(1, -1), g1.reshape(1, -1),
      bt1.reshape(1, -1), W2.astype(bf), b2.reshape(1, -1),
      g2.reshape(1, -1), bt2.reshape(1, -1), W3.astype(bf),
      b3.reshape(1, -1), W4.astype(bf), b4.reshape(1, -1))
    return out
